# baseline (speedup 1.0000x reference)
import functools

import jax
import jax.numpy as jnp
from jax import lax
from jax.experimental import pallas as pl
from jax.experimental.pallas import tpu as pltpu
from jax.experimental.pallas import tpu_sc as plsc

_NC = 2
_NS = 16
_NW = _NC * _NS
_LANES = 16

_B = 4096
_L = 50
_D = 128
_S = 4
_BS = _B // _S
_RPW = _BS // _NW
_R = 4
_CH = _RPW // _R
_PAIRS = _CH // 2


def _tanh_vec(x):
    e = jnp.exp(x + x)
    return 1.0 - 2.0 / (e + 1.0)


def _body(ids_hbm, emb_hbm, glove_hbm, out_hbm, idx_all, emb2, glove2,
          gsem0, gsem1, ssem0, ssem1, *, s0):
    wid = lax.axis_index("s") * _NC + lax.axis_index("c")
    r0in = s0 + wid * _RPW
    r0out = wid * _RPW

    pltpu.sync_copy(ids_hbm.at[pl.ds(r0in, _RPW), :], idx_all)

    def idx_ref(c, r):
        return idx_all.at[c * _R + r]

    def start_gathers(c, b, gsem):
        for r in range(_R):
            pltpu.async_copy(emb_hbm.at[idx_ref(c, r)], emb2.at[b, r], gsem)
            pltpu.async_copy(glove_hbm.at[idx_ref(c, r)], glove2.at[b, r],
                             gsem)

    def wait_gathers(b, gsem):
        for r in range(_R):
            pltpu.make_async_copy(emb_hbm.at[idx_ref(0, r)], emb2.at[b, r],
                                  gsem).wait()
            pltpu.make_async_copy(glove_hbm.at[idx_ref(0, r)],
                                  glove2.at[b, r], gsem).wait()

    def start_stores(c, b, ssem):
        base = r0out + c * _R
        pltpu.async_copy(
            emb2.at[b], out_hbm.at[pl.ds(base, _R), :, pl.ds(0, _D)], ssem)
        pltpu.async_copy(
            glove2.at[b], out_hbm.at[pl.ds(base, _R), :, pl.ds(_D, _D)], ssem)

    def wait_stores(b, ssem):
        pltpu.make_async_copy(
            emb2.at[b], out_hbm.at[pl.ds(0, _R), :, pl.ds(0, _D)],
            ssem).wait()
        pltpu.make_async_copy(
            emb2.at[b], out_hbm.at[pl.ds(0, _R), :, pl.ds(_D, _D)],
            ssem).wait()

    def tanh_chunk(b):
        for r in range(_R):
            def l_body(l, carry):
                for j in range(_D // _LANES):
                    sl = pl.ds(j * _LANES, _LANES)
                    emb2[b, r, l, sl] = _tanh_vec(emb2[b, r, l, sl])
                return carry

            lax.fori_loop(0, _L, l_body, 0)

    start_gathers(0, 0, gsem0)

    def pair_body(i, carry):
        c0 = i * 2
        wait_gathers(0, gsem0)

        @pl.when(i > 0)
        def _():
            wait_stores(1, ssem1)

        start_gathers(c0 + 1, 1, gsem1)
        tanh_chunk(0)
        start_stores(c0, 0, ssem0)

        wait_gathers(1, gsem1)

        @pl.when(i < _PAIRS - 1)
        def _():
            wait_stores(0, ssem0)
            start_gathers(c0 + 2, 0, gsem0)

        tanh_chunk(1)
        start_stores(c0 + 1, 1, ssem1)
        return carry

    lax.fori_loop(0, _PAIRS, pair_body, 0)
    wait_stores(0, ssem0)
    wait_stores(1, ssem1)


@jax.jit
def _run(ids, emb_table, glove_table):
    mesh = plsc.VectorSubcoreMesh(
        core_axis_name="c", subcore_axis_name="s",
        num_cores=_NC, num_subcores=_NS)
    outs = []
    for s in range(_S):
        f = pl.kernel(
            functools.partial(_body, s0=s * _BS),
            out_type=jax.ShapeDtypeStruct((_BS, _L, 2 * _D), jnp.float32),
            mesh=mesh,
            scratch_types=[
                pltpu.VMEM((_RPW, _L), jnp.int32),
                pltpu.VMEM((2, _R, _L, _D), jnp.float32),
                pltpu.VMEM((2, _R, _L, _D), jnp.float32),
                pltpu.SemaphoreType.DMA,
                pltpu.SemaphoreType.DMA,
                pltpu.SemaphoreType.DMA,
                pltpu.SemaphoreType.DMA,
            ],
        )
        outs.append(f(ids, emb_table, glove_table))
    return jnp.concatenate(outs, axis=0)


def kernel(input_ids, emb_table, glove_table):
    return _run(input_ids.astype(jnp.int32), emb_table, glove_table)


# speedup vs baseline: 1.6210x; 1.6210x over previous
import jax
import jax.numpy as jnp
from jax import lax
from jax.experimental import pallas as pl
from jax.experimental.pallas import tpu as pltpu
from jax.experimental.pallas import tpu_sc as plsc

_NC = 2
_NS = 16
_NW = _NC * _NS
_LANES = 16

_B = 4096
_L = 50
_D = 128
_RPW = _B // _NW
_R = 4
_CH = _RPW // _R
_PAIRS = _CH // 2


def _tanh_vec(x):
    e = jnp.exp(x + x)
    return 1.0 - 2.0 / (e + 1.0)


def _body(ids_hbm, emb_hbm, glove_hbm, out_hbm, idx_all, emb2, glove2,
          gsem0, gsem1, ssem0, ssem1):
    wid = lax.axis_index("s") * _NC + lax.axis_index("c")
    r0w = wid * _RPW

    pltpu.sync_copy(ids_hbm.at[pl.ds(r0w, _RPW), :], idx_all)

    def idx_ref(c, r):
        return idx_all.at[c * _R + r]

    def start_gathers(c, b, gsem):
        for r in range(_R):
            pltpu.async_copy(emb_hbm.at[idx_ref(c, r)], emb2.at[b, r], gsem)
            pltpu.async_copy(glove_hbm.at[idx_ref(c, r)], glove2.at[b, r],
                             gsem)

    def wait_gathers(b, gsem):
        for r in range(_R):
            pltpu.make_async_copy(emb_hbm.at[idx_ref(0, r)], emb2.at[b, r],
                                  gsem).wait()
            pltpu.make_async_copy(glove_hbm.at[idx_ref(0, r)],
                                  glove2.at[b, r], gsem).wait()

    def start_stores(c, b, ssem):
        base = r0w + c * _R
        pltpu.async_copy(
            emb2.at[b], out_hbm.at[pl.ds(base, _R), :, pl.ds(0, _D)], ssem)
        pltpu.async_copy(
            glove2.at[b], out_hbm.at[pl.ds(base, _R), :, pl.ds(_D, _D)], ssem)

    def wait_stores(b, ssem):
        pltpu.make_async_copy(
            emb2.at[b], out_hbm.at[pl.ds(0, _R), :, pl.ds(0, _D)],
            ssem).wait()
        pltpu.make_async_copy(
            glove2.at[b], out_hbm.at[pl.ds(0, _R), :, pl.ds(_D, _D)],
            ssem).wait()

    def tanh_chunk(b):
        for r in range(_R):
            def l_body(l, carry):
                for j in range(_D // _LANES):
                    sl = pl.ds(j * _LANES, _LANES)
                    emb2[b, r, l, sl] = _tanh_vec(emb2[b, r, l, sl])
                return carry

            lax.fori_loop(0, _L, l_body, 0)

    start_gathers(0, 0, gsem0)

    def pair_body(i, carry):
        c0 = i * 2
        wait_gathers(0, gsem0)

        @pl.when(i > 0)
        def _():
            wait_stores(1, ssem1)

        start_gathers(c0 + 1, 1, gsem1)
        tanh_chunk(0)
        start_stores(c0, 0, ssem0)

        wait_gathers(1, gsem1)

        @pl.when(i < _PAIRS - 1)
        def _():
            wait_stores(0, ssem0)
            start_gathers(c0 + 2, 0, gsem0)

        tanh_chunk(1)
        start_stores(c0 + 1, 1, ssem1)
        return carry

    lax.fori_loop(0, _PAIRS, pair_body, 0)
    wait_stores(0, ssem0)
    wait_stores(1, ssem1)


@jax.jit
def _run(ids, emb_table, glove_table):
    mesh = plsc.VectorSubcoreMesh(
        core_axis_name="c", subcore_axis_name="s",
        num_cores=_NC, num_subcores=_NS)
    f = pl.kernel(
        _body,
        out_type=jax.ShapeDtypeStruct((_B, _L, 2 * _D), jnp.float32),
        mesh=mesh,
        scratch_types=[
            pltpu.VMEM((_RPW, _L), jnp.int32),
            pltpu.VMEM((2, _R, _L, _D), jnp.float32),
            pltpu.VMEM((2, _R, _L, _D), jnp.float32),
            pltpu.SemaphoreType.DMA,
            pltpu.SemaphoreType.DMA,
            pltpu.SemaphoreType.DMA,
            pltpu.SemaphoreType.DMA,
        ],
        compiler_params=pltpu.CompilerParams(use_tc_tiling_on_sc=True),
    )
    return f(ids, emb_table, glove_table)


def kernel(input_ids, emb_table, glove_table):
    return _run(input_ids.astype(jnp.int32), emb_table, glove_table)


# speedup vs baseline: 2.5726x; 1.5870x over previous
import jax
import jax.numpy as jnp
from jax import lax
from jax.experimental import pallas as pl
from jax.experimental.pallas import tpu as pltpu
from jax.experimental.pallas import tpu_sc as plsc

_NC = 2
_NS = 16
_NW = _NC * _NS
_LANES = 16

_B = 4096
_L = 50
_D = 128
_BPW = _B // _NW


def _tanh_vec(x):
    e = jnp.exp(x + x)
    return 1.0 - 2.0 / (e + 1.0)


def _body(ids_t_hbm, emb_hbm, glove_hbm, out_hbm, idx_t, ebuf, gbuf,
          gsem0, gsem1, ssem0, ssem1):
    wid = lax.axis_index("s") * _NC + lax.axis_index("c")
    bt = wid * _BPW

    pltpu.sync_copy(ids_t_hbm.at[:, pl.ds(bt, _BPW)], idx_t)

    gsems = (gsem0, gsem1)
    ssems = (ssem0, ssem1)

    def start_gathers(l, b):
        pltpu.async_copy(emb_hbm.at[idx_t.at[l]], ebuf.at[b], gsems[b])
        pltpu.async_copy(glove_hbm.at[idx_t.at[l]], gbuf.at[b], gsems[b])

    def wait_gathers(b):
        pltpu.make_async_copy(emb_hbm.at[idx_t.at[0]], ebuf.at[b],
                              gsems[b]).wait()
        pltpu.make_async_copy(glove_hbm.at[idx_t.at[0]], gbuf.at[b],
                              gsems[b]).wait()

    def start_stores(l, b):
        pltpu.async_copy(
            ebuf.at[b], out_hbm.at[l, pl.ds(bt, _BPW), pl.ds(0, _D)],
            ssems[b])
        pltpu.async_copy(
            gbuf.at[b], out_hbm.at[l, pl.ds(bt, _BPW), pl.ds(_D, _D)],
            ssems[b])

    def wait_stores(b):
        pltpu.make_async_copy(
            ebuf.at[b], out_hbm.at[0, pl.ds(bt, _BPW), pl.ds(0, _D)],
            ssems[b]).wait()
        pltpu.make_async_copy(
            gbuf.at[b], out_hbm.at[0, pl.ds(bt, _BPW), pl.ds(_D, _D)],
            ssems[b]).wait()

    def tanh_chunk(b):
        def r_body(r, carry):
            for j in range(_D // _LANES):
                sl = pl.ds(j * _LANES, _LANES)
                ebuf[b, r, sl] = _tanh_vec(ebuf[b, r, sl])
            return carry

        lax.fori_loop(0, _BPW, r_body, 0)

    start_gathers(0, 0)
    for l in range(_L):
        b = l % 2
        wait_gathers(b)
        if l + 1 < _L:
            if l >= 1:
                wait_stores(1 - b)
            start_gathers(l + 1, 1 - b)
        tanh_chunk(b)
        start_stores(l, b)
    wait_stores(0)
    wait_stores(1)


@jax.jit
def _run(ids_t, emb_table, glove_table):
    mesh = plsc.VectorSubcoreMesh(
        core_axis_name="c", subcore_axis_name="s",
        num_cores=_NC, num_subcores=_NS)
    f = pl.kernel(
        _body,
        out_type=jax.ShapeDtypeStruct((_L, _B, 2 * _D), jnp.float32),
        mesh=mesh,
        scratch_types=[
            pltpu.VMEM((_L, _BPW), jnp.int32),
            pltpu.VMEM((2, _BPW, _D), jnp.float32),
            pltpu.VMEM((2, _BPW, _D), jnp.float32),
            pltpu.SemaphoreType.DMA,
            pltpu.SemaphoreType.DMA,
            pltpu.SemaphoreType.DMA,
            pltpu.SemaphoreType.DMA,
        ],
    )
    out_t = f(ids_t, emb_table, glove_table)
    return jnp.swapaxes(out_t, 0, 1)


def kernel(input_ids, emb_table, glove_table):
    ids_t = jnp.swapaxes(input_ids, 0, 1).astype(jnp.int32)
    return _run(ids_t, emb_table, glove_table)


# speedup vs baseline: 3.2552x; 1.2653x over previous
import jax
import jax.numpy as jnp
from jax import lax
from jax.experimental import pallas as pl
from jax.experimental.pallas import tpu as pltpu
from jax.experimental.pallas import tpu_sc as plsc

_NC = 2
_NS = 16
_NW = _NC * _NS
_LANES = 16

_B = 4096
_L = 50
_D = 128
_BPW = _B // _NW


def _tanh_vec(x):
    e = jnp.exp(x + x)
    return 1.0 - 2.0 / (e + 1.0)


def _body(ids_t_hbm, emb_hbm, glove_hbm, out_hbm, idx_t, ebuf, gbuf,
          gsem0, gsem1, gsem2, ssem0, ssem1, ssem2):
    wid = lax.axis_index("s") * _NC + lax.axis_index("c")
    bt = wid * _BPW

    pltpu.sync_copy(ids_t_hbm.at[:, pl.ds(bt, _BPW)], idx_t)

    gsems = (gsem0, gsem1, gsem2)
    ssems = (ssem0, ssem1, ssem2)

    def start_gathers(l, b):
        pltpu.async_copy(emb_hbm.at[idx_t.at[l]], ebuf.at[b], gsems[b])
        pltpu.async_copy(glove_hbm.at[idx_t.at[l]], gbuf.at[b], gsems[b])

    def wait_gathers(b):
        pltpu.make_async_copy(emb_hbm.at[idx_t.at[0]], ebuf.at[b],
                              gsems[b]).wait()
        pltpu.make_async_copy(glove_hbm.at[idx_t.at[0]], gbuf.at[b],
                              gsems[b]).wait()

    def start_stores(l, b):
        pltpu.async_copy(
            ebuf.at[b], out_hbm.at[l, pl.ds(bt, _BPW), pl.ds(0, _D)],
            ssems[b])
        pltpu.async_copy(
            gbuf.at[b], out_hbm.at[l, pl.ds(bt, _BPW), pl.ds(_D, _D)],
            ssems[b])

    def wait_stores(b):
        pltpu.make_async_copy(
            ebuf.at[b], out_hbm.at[0, pl.ds(bt, _BPW), pl.ds(0, _D)],
            ssems[b]).wait()
        pltpu.make_async_copy(
            gbuf.at[b], out_hbm.at[0, pl.ds(bt, _BPW), pl.ds(_D, _D)],
            ssems[b]).wait()

    def tanh_chunk(b):
        def r_body(r, carry):
            for rr in range(2):
                for j in range(_D // _LANES):
                    sl = pl.ds(j * _LANES, _LANES)
                    ebuf[b, r * 2 + rr, sl] = _tanh_vec(ebuf[b, r * 2 + rr,
                                                             sl])
            return carry

        lax.fori_loop(0, _BPW // 2, r_body, 0)

    start_gathers(0, 0)
    start_gathers(1, 1)
    for l in range(_L):
        b = l % 3
        wait_gathers(b)
        if l + 2 < _L:
            if l >= 1:
                wait_stores((l + 2) % 3)
            start_gathers(l + 2, (l + 2) % 3)
        tanh_chunk(b)
        start_stores(l, b)
    for l in range(_L - 3, _L):
        wait_stores(l % 3)


@jax.jit
def _run(ids_t, emb_table, glove_table):
    mesh = plsc.VectorSubcoreMesh(
        core_axis_name="c", subcore_axis_name="s",
        num_cores=_NC, num_subcores=_NS)
    f = pl.kernel(
        _body,
        out_type=jax.ShapeDtypeStruct((_L, _B, 2 * _D), jnp.float32),
        mesh=mesh,
        scratch_types=[
            pltpu.VMEM((_L, _BPW), jnp.int32),
            pltpu.VMEM((3, _BPW, _D), jnp.float32),
            pltpu.VMEM((3, _BPW, _D), jnp.float32),
            pltpu.SemaphoreType.DMA,
            pltpu.SemaphoreType.DMA,
            pltpu.SemaphoreType.DMA,
            pltpu.SemaphoreType.DMA,
            pltpu.SemaphoreType.DMA,
            pltpu.SemaphoreType.DMA,
        ],
    )
    out_t = f(ids_t, emb_table, glove_table)
    return jnp.swapaxes(out_t, 0, 1)


def kernel(input_ids, emb_table, glove_table):
    ids_t = jnp.swapaxes(input_ids, 0, 1).astype(jnp.int32)
    return _run(ids_t, emb_table, glove_table)
